# Initial kernel scaffold; baseline (speedup 1.0000x reference)
import functools

import jax
import jax.numpy as jnp
from jax import lax
from jax.experimental import pallas as pl
from jax.experimental.pallas import tpu as pltpu
from jax.experimental.pallas import tpu_sc as plsc

N = 10000
E = 320000
D_IN = 128
D = 64

NC = 2
NS = 16
NW = NC * NS
B = 128
NB = 79
E_PAD = NW * NB * B
NPAD = 10240
RS = NPAD // NS
MBLK = 256
GRID_M = NPAD // MBLK

_mesh = plsc.VectorSubcoreMesh(core_axis_name="c", subcore_axis_name="s")


@functools.partial(
    pl.kernel,
    out_type=jax.ShapeDtypeStruct((NC, NPAD), jnp.float32),
    mesh=_mesh,
    scratch_types=[
        pltpu.VMEM((B,), jnp.int32),
        pltpu.VMEM((B,), jnp.float32),
        pltpu.VMEM((RS,), jnp.float32),
        pltpu.VMEM_SHARED((NPAD,), jnp.float32),
    ],
)
def _deg(dst_hbm, ew_hbm, out_hbm, idx_v, val_v, zero_v, acc):
    c = lax.axis_index("c")
    s = lax.axis_index("s")
    wid = c * NS + s

    def zfill(i, _):
        zero_v[pl.ds(i * 16, 16)] = jnp.zeros((16,), jnp.float32)
        return 0

    lax.fori_loop(0, RS // 16, zfill, 0)
    pltpu.sync_copy(zero_v, acc.at[pl.ds(s * RS, RS)])
    plsc.subcore_barrier()

    base = wid * NB * B

    def body(b, _):
        off = base + b * B
        pltpu.sync_copy(dst_hbm.at[pl.ds(off, B)], idx_v)
        pltpu.sync_copy(ew_hbm.at[pl.ds(off, B)], val_v)
        pltpu.sync_copy(val_v, acc.at[idx_v], add=True)
        return 0

    lax.fori_loop(0, NB, body, 0)
    plsc.subcore_barrier()
    pltpu.sync_copy(acc.at[pl.ds(s * RS, RS)], out_hbm.at[c, pl.ds(s * RS, RS)])


@functools.partial(
    pl.kernel,
    out_type=jax.ShapeDtypeStruct((NC, NPAD, D), jnp.float32),
    mesh=_mesh,
    scratch_types=[
        pltpu.VMEM((B,), jnp.int32),
        pltpu.VMEM((B,), jnp.int32),
        pltpu.VMEM((B,), jnp.float32),
        pltpu.VMEM((B, D), jnp.float32),
        pltpu.VMEM((64, D), jnp.float32),
        pltpu.VMEM_SHARED((NPAD, D), jnp.float32),
        pltpu.SemaphoreType.DMA,
    ],
)
def _agg(y_hbm, src_hbm, dst_hbm, ew_hbm, out_hbm,
         srcb, dstb, ewb, rows, zbuf, acc, sem):
    c = lax.axis_index("c")
    s = lax.axis_index("s")
    wid = c * NS + s

    for r in range(64):
        for j in range(D // 16):
            zbuf[r, pl.ds(j * 16, 16)] = jnp.zeros((16,), jnp.float32)
    for t in range(RS // 64):
        pltpu.sync_copy(zbuf, acc.at[pl.ds(s * RS + t * 64, 64)])
    plsc.subcore_barrier()

    base = wid * NB * B

    def body(b, _):
        off = base + b * B
        pltpu.sync_copy(src_hbm.at[pl.ds(off, B)], srcb)
        pltpu.sync_copy(ew_hbm.at[pl.ds(off, B)], ewb)
        pltpu.sync_copy(dst_hbm.at[pl.ds(off, B)], dstb)
        pltpu.async_copy(y_hbm.at[srcb], rows, sem).wait()
        for e in range(B):
            w = plsc.load_gather(ewb, [jnp.full((16,), e, jnp.int32)])
            for j in range(D // 16):
                rows[e, pl.ds(j * 16, 16)] = rows[e, pl.ds(j * 16, 16)] * w
        pltpu.sync_copy(rows, acc.at[dstb], add=True)
        return 0

    lax.fori_loop(0, NB, body, 0)
    plsc.subcore_barrier()
    pltpu.sync_copy(acc.at[pl.ds(s * RS, RS)], out_hbm.at[c, pl.ds(s * RS, RS)])


def _tc1_body(degT_ref, x_ref, W1_ref, dis_ref, y_ref):
    deg = 1.0 + degT_ref[:, 0:1] + degT_ref[:, 1:2]
    dis = jnp.where(deg > 0, lax.rsqrt(deg), 0.0)
    dis_ref[...] = dis
    y_ref[...] = dis * jnp.dot(x_ref[...], W1_ref[...],
                               preferred_element_type=jnp.float32)


_tc1 = pl.pallas_call(
    _tc1_body,
    grid=(GRID_M,),
    in_specs=[
        pl.BlockSpec((MBLK, 2), lambda i: (i, 0)),
        pl.BlockSpec((MBLK, D_IN), lambda i: (i, 0)),
        pl.BlockSpec((D_IN, D), lambda i: (0, 0)),
    ],
    out_specs=[
        pl.BlockSpec((MBLK, 1), lambda i: (i, 0)),
        pl.BlockSpec((MBLK, D), lambda i: (i, 0)),
    ],
    out_shape=[
        jax.ShapeDtypeStruct((NPAD, 1), jnp.float32),
        jax.ShapeDtypeStruct((NPAD, D), jnp.float32),
    ],
)


def _tclayer_body(s0_ref, s1_ref, yp_ref, dis_ref, b_ref, W_ref, y_ref):
    dis = dis_ref[...]
    s = s0_ref[...] + s1_ref[...] + yp_ref[...]
    h = jax.nn.relu(dis * s + b_ref[...])
    y_ref[...] = dis * jnp.dot(h, W_ref[...], preferred_element_type=jnp.float32)


_tclayer = pl.pallas_call(
    _tclayer_body,
    grid=(GRID_M,),
    in_specs=[
        pl.BlockSpec((MBLK, D), lambda i: (i, 0)),
        pl.BlockSpec((MBLK, D), lambda i: (i, 0)),
        pl.BlockSpec((MBLK, D), lambda i: (i, 0)),
        pl.BlockSpec((MBLK, 1), lambda i: (i, 0)),
        pl.BlockSpec((1, D), lambda i: (0, 0)),
        pl.BlockSpec((D, D), lambda i: (0, 0)),
    ],
    out_specs=pl.BlockSpec((MBLK, D), lambda i: (i, 0)),
    out_shape=jax.ShapeDtypeStruct((NPAD, D), jnp.float32),
)


def _tcfinal_body(s0_ref, s1_ref, yp_ref, dis_ref, b5_ref, fc1W_ref, fc1b_ref,
                  fc2W_ref, fc2b_ref, o_ref):
    dis = dis_ref[...]
    s = s0_ref[...] + s1_ref[...] + yp_ref[...]
    h = jax.nn.relu(dis * s + b5_ref[...])
    h = jax.nn.relu(jnp.dot(h, fc1W_ref[...], preferred_element_type=jnp.float32)
                    + fc1b_ref[...])
    o_ref[...] = jnp.dot(h, fc2W_ref[...], preferred_element_type=jnp.float32) \
        + fc2b_ref[...]


_tcfinal = pl.pallas_call(
    _tcfinal_body,
    grid=(GRID_M,),
    in_specs=[
        pl.BlockSpec((MBLK, D), lambda i: (i, 0)),
        pl.BlockSpec((MBLK, D), lambda i: (i, 0)),
        pl.BlockSpec((MBLK, D), lambda i: (i, 0)),
        pl.BlockSpec((MBLK, 1), lambda i: (i, 0)),
        pl.BlockSpec((1, D), lambda i: (0, 0)),
        pl.BlockSpec((D, D), lambda i: (0, 0)),
        pl.BlockSpec((1, D), lambda i: (0, 0)),
        pl.BlockSpec((D, 128), lambda i: (0, 0)),
        pl.BlockSpec((1, 128), lambda i: (0, 0)),
    ],
    out_specs=pl.BlockSpec((MBLK, 128), lambda i: (i, 0)),
    out_shape=jax.ShapeDtypeStruct((NPAD, 128), jnp.float32),
)


def kernel(x, edge_index, edge_attr, W1, b1, W2, b2, W3, b3, W4, b4, W5, b5,
           fc1_W, fc1_b, fc2_W, fc2_b):
    src = edge_index[0].astype(jnp.int32)
    dst = edge_index[1].astype(jnp.int32)
    ew = edge_attr.astype(jnp.float32)

    npad_rows = E_PAD - E
    padidx = N + (jnp.arange(npad_rows, dtype=jnp.int32) % (NPAD - N))
    srcp = jnp.concatenate([src, padidx])
    dstp = jnp.concatenate([dst, padidx])
    ewp = jnp.concatenate([ew, jnp.zeros((npad_rows,), jnp.float32)])
    xp = jnp.pad(x, ((0, NPAD - N), (0, 0)))

    deg01 = _deg(dstp, ewp)
    degT = deg01.T
    dis, y = _tc1(degT, xp, W1)

    bs = [b1.reshape(1, D), b2.reshape(1, D), b3.reshape(1, D),
          b4.reshape(1, D), b5.reshape(1, D)]
    Ws = [W2, W3, W4, W5]

    for li in range(4):
        sp = _agg(y, srcp, dstp, ewp)
        y = _tclayer(sp[0], sp[1], y, dis, bs[li], Ws[li])
    sp = _agg(y, srcp, dstp, ewp)

    fc2_Wp = jnp.pad(fc2_W, ((0, 0), (0, 128 - fc2_W.shape[1])))
    fc2_bp = jnp.pad(fc2_b.reshape(1, -1), ((0, 0), (0, 128 - fc2_b.shape[0])))
    out = _tcfinal(sp[0], sp[1], y, dis, bs[4], fc1_W,
                   fc1_b.reshape(1, D), fc2_Wp, fc2_bp)
    return out[:N, :1]


# speedup vs baseline: 8.6971x; 8.6971x over previous
import functools

import jax
import jax.numpy as jnp
from jax import lax
from jax.experimental import pallas as pl
from jax.experimental.pallas import tpu as pltpu
from jax.experimental.pallas import tpu_sc as plsc

N = 10000
E = 320000
D_IN = 128
D = 64

NC = 2
NS = 16
NW = NC * NS
B = 128
NB = 79
E_PAD = NW * NB * B
NPAD = 10240
RS = NPAD // NS
MBLK = 256
GRID_M = NPAD // MBLK

_mesh = plsc.VectorSubcoreMesh(core_axis_name="c", subcore_axis_name="s")


@functools.partial(
    pl.kernel,
    out_type=jax.ShapeDtypeStruct((NC, NPAD), jnp.float32),
    mesh=_mesh,
    scratch_types=[
        pltpu.VMEM((B,), jnp.int32),
        pltpu.VMEM((B,), jnp.float32),
        pltpu.VMEM((RS,), jnp.float32),
        pltpu.VMEM_SHARED((NPAD,), jnp.float32),
    ],
    compiler_params=pltpu.CompilerParams(needs_layout_passes=False,
                                         use_tc_tiling_on_sc=False),
)
def _deg(dst_hbm, ew_hbm, out_hbm, idx_v, val_v, zero_v, acc):
    c = lax.axis_index("c")
    s = lax.axis_index("s")
    wid = c * NS + s

    def zfill(i, _):
        zero_v[pl.ds(i * 16, 16)] = jnp.zeros((16,), jnp.float32)
        return 0

    lax.fori_loop(0, RS // 16, zfill, 0)
    pltpu.sync_copy(zero_v, acc.at[pl.ds(s * RS, RS)])
    plsc.subcore_barrier()

    base = wid * NB * B

    def body(b, _):
        off = base + b * B
        pltpu.sync_copy(dst_hbm.at[pl.ds(off, B)], idx_v)
        pltpu.sync_copy(ew_hbm.at[pl.ds(off, B)], val_v)
        pltpu.sync_copy(val_v, acc.at[idx_v], add=True)
        return 0

    lax.fori_loop(0, NB, body, 0)
    plsc.subcore_barrier()
    pltpu.sync_copy(acc.at[pl.ds(s * RS, RS)], out_hbm.at[c, pl.ds(s * RS, RS)])


@functools.partial(
    pl.kernel,
    out_type=jax.ShapeDtypeStruct((NC, NPAD, D), jnp.float32),
    mesh=_mesh,
    scratch_types=[
        pltpu.VMEM((B,), jnp.int32),
        pltpu.VMEM((B,), jnp.int32),
        pltpu.VMEM((B,), jnp.float32),
        pltpu.VMEM((B, D), jnp.float32),
        pltpu.VMEM((64, D), jnp.float32),
        pltpu.VMEM_SHARED((NPAD, D), jnp.float32),
        pltpu.SemaphoreType.DMA,
    ],
    compiler_params=pltpu.CompilerParams(needs_layout_passes=False,
                                         use_tc_tiling_on_sc=False),
)
def _agg(y_hbm, src_hbm, dst_hbm, ew_hbm, out_hbm,
         srcb, dstb, ewb, rows, zbuf, acc, sem):
    c = lax.axis_index("c")
    s = lax.axis_index("s")
    wid = c * NS + s

    for r in range(64):
        for j in range(D // 16):
            zbuf[r, pl.ds(j * 16, 16)] = jnp.zeros((16,), jnp.float32)
    for t in range(RS // 64):
        pltpu.sync_copy(zbuf, acc.at[pl.ds(s * RS + t * 64, 64)])
    plsc.subcore_barrier()

    base = wid * NB * B

    def body(b, _):
        off = base + b * B
        pltpu.sync_copy(src_hbm.at[pl.ds(off, B)], srcb)
        pltpu.sync_copy(ew_hbm.at[pl.ds(off, B)], ewb)
        pltpu.sync_copy(dst_hbm.at[pl.ds(off, B)], dstb)
        pltpu.async_copy(y_hbm.at[srcb], rows, sem).wait()
        for e in range(B):
            if e == 0:
                ew0 = ewb[pl.ds(0, 16)]
                lane = lax.broadcasted_iota(jnp.int32, (16,), 0)
                w0 = jnp.sum(jnp.where(lane == 0, ew0, 0.0))
                w = jnp.full((16,), w0)
            else:
                w = plsc.load_gather(ewb, [jnp.full((16,), e, jnp.int32)])
            for j in range(D // 16):
                rows[e, pl.ds(j * 16, 16)] = rows[e, pl.ds(j * 16, 16)] * w
        pltpu.sync_copy(rows, acc.at[dstb], add=True)
        return 0

    lax.fori_loop(0, NB, body, 0)
    plsc.subcore_barrier()
    pltpu.sync_copy(acc.at[pl.ds(s * RS, RS)], out_hbm.at[c, pl.ds(s * RS, RS)])


def _tc1_body(degT_ref, x_ref, W1_ref, dis_ref, y_ref):
    deg = 1.0 + degT_ref[:, 0:1] + degT_ref[:, 1:2]
    dis = jnp.where(deg > 0, lax.rsqrt(deg), 0.0)
    dis_ref[...] = dis
    y_ref[...] = dis * jnp.dot(x_ref[...], W1_ref[...],
                               preferred_element_type=jnp.float32,
                               precision=lax.Precision.DEFAULT)


_tc1 = pl.pallas_call(
    _tc1_body,
    grid=(GRID_M,),
    in_specs=[
        pl.BlockSpec((MBLK, 2), lambda i: (i, 0)),
        pl.BlockSpec((MBLK, D_IN), lambda i: (i, 0)),
        pl.BlockSpec((D_IN, D), lambda i: (0, 0)),
    ],
    out_specs=[
        pl.BlockSpec((MBLK, 1), lambda i: (i, 0)),
        pl.BlockSpec((MBLK, D), lambda i: (i, 0)),
    ],
    out_shape=[
        jax.ShapeDtypeStruct((NPAD, 1), jnp.float32),
        jax.ShapeDtypeStruct((NPAD, D), jnp.float32),
    ],
)


def _tclayer_body(s0_ref, s1_ref, yp_ref, dis_ref, b_ref, W_ref, y_ref):
    dis = dis_ref[...]
    s = s0_ref[...] + s1_ref[...] + yp_ref[...]
    h = jax.nn.relu(dis * s + b_ref[...])
    y_ref[...] = dis * jnp.dot(h, W_ref[...], preferred_element_type=jnp.float32,
                               precision=lax.Precision.DEFAULT)


_tclayer = pl.pallas_call(
    _tclayer_body,
    grid=(GRID_M,),
    in_specs=[
        pl.BlockSpec((MBLK, D), lambda i: (i, 0)),
        pl.BlockSpec((MBLK, D), lambda i: (i, 0)),
        pl.BlockSpec((MBLK, D), lambda i: (i, 0)),
        pl.BlockSpec((MBLK, 1), lambda i: (i, 0)),
        pl.BlockSpec((1, D), lambda i: (0, 0)),
        pl.BlockSpec((D, D), lambda i: (0, 0)),
    ],
    out_specs=pl.BlockSpec((MBLK, D), lambda i: (i, 0)),
    out_shape=jax.ShapeDtypeStruct((NPAD, D), jnp.float32),
)


def _tcfinal_body(s0_ref, s1_ref, yp_ref, dis_ref, b5_ref, fc1W_ref, fc1b_ref,
                  fc2W_ref, fc2b_ref, o_ref):
    dis = dis_ref[...]
    s = s0_ref[...] + s1_ref[...] + yp_ref[...]
    h = jax.nn.relu(dis * s + b5_ref[...])
    h = jax.nn.relu(jnp.dot(h, fc1W_ref[...], preferred_element_type=jnp.float32,
                               precision=lax.Precision.DEFAULT)
                    + fc1b_ref[...])
    o_ref[...] = jnp.dot(h, fc2W_ref[...], preferred_element_type=jnp.float32,
                               precision=lax.Precision.DEFAULT) \
        + fc2b_ref[...]


_tcfinal = pl.pallas_call(
    _tcfinal_body,
    grid=(GRID_M,),
    in_specs=[
        pl.BlockSpec((MBLK, D), lambda i: (i, 0)),
        pl.BlockSpec((MBLK, D), lambda i: (i, 0)),
        pl.BlockSpec((MBLK, D), lambda i: (i, 0)),
        pl.BlockSpec((MBLK, 1), lambda i: (i, 0)),
        pl.BlockSpec((1, D), lambda i: (0, 0)),
        pl.BlockSpec((D, D), lambda i: (0, 0)),
        pl.BlockSpec((1, D), lambda i: (0, 0)),
        pl.BlockSpec((D, 128), lambda i: (0, 0)),
        pl.BlockSpec((1, 128), lambda i: (0, 0)),
    ],
    out_specs=pl.BlockSpec((MBLK, 128), lambda i: (i, 0)),
    out_shape=jax.ShapeDtypeStruct((NPAD, 128), jnp.float32),
)


def kernel(x, edge_index, edge_attr, W1, b1, W2, b2, W3, b3, W4, b4, W5, b5,
           fc1_W, fc1_b, fc2_W, fc2_b):
    src = edge_index[0].astype(jnp.int32)
    dst = edge_index[1].astype(jnp.int32)
    ew = edge_attr.astype(jnp.float32)

    npad_rows = E_PAD - E
    padidx = N + (jnp.arange(npad_rows, dtype=jnp.int32) % (NPAD - N))
    srcp = jnp.concatenate([src, padidx])
    dstp = jnp.concatenate([dst, padidx])
    ewp = jnp.concatenate([ew, jnp.zeros((npad_rows,), jnp.float32)])
    xp = jnp.pad(x, ((0, NPAD - N), (0, 0)))

    deg01 = _deg(dstp, ewp)
    degT = deg01.T
    dis, y = _tc1(degT, xp, W1)

    bs = [b1.reshape(1, D), b2.reshape(1, D), b3.reshape(1, D),
          b4.reshape(1, D), b5.reshape(1, D)]
    Ws = [W2, W3, W4, W5]

    for li in range(4):
        sp = _agg(y, srcp, dstp, ewp)
        y = _tclayer(sp[0], sp[1], y, dis, bs[li], Ws[li])
    sp = _agg(y, srcp, dstp, ewp)

    fc2_Wp = jnp.pad(fc2_W, ((0, 0), (0, 128 - fc2_W.shape[1])))
    fc2_bp = jnp.pad(fc2_b.reshape(1, -1), ((0, 0), (0, 128 - fc2_b.shape[0])))
    out = _tcfinal(sp[0], sp[1], y, dis, bs[4], fc1_W,
                   fc1_b.reshape(1, D), fc2_Wp, fc2_bp)
    return out[:N, :1]


# speedup vs baseline: 16.1532x; 1.8573x over previous
import functools

import jax
import jax.numpy as jnp
from jax import lax
from jax.experimental import pallas as pl
from jax.experimental.pallas import tpu as pltpu
from jax.experimental.pallas import tpu_sc as plsc

N = 10000
E = 320000
D_IN = 128
D = 64

NC = 2
NS = 16
NW = NC * NS
B = 128
NB = 80
E_PAD = NW * NB * B
NBUF = 4
NPAD = 10240
RS = NPAD // NS
MBLK = 256
GRID_M = NPAD // MBLK

_mesh = plsc.VectorSubcoreMesh(core_axis_name="c", subcore_axis_name="s")


@functools.partial(
    pl.kernel,
    out_type=jax.ShapeDtypeStruct((NC, NPAD), jnp.float32),
    mesh=_mesh,
    scratch_types=[
        pltpu.VMEM((B,), jnp.int32),
        pltpu.VMEM((B,), jnp.float32),
        pltpu.VMEM((RS,), jnp.float32),
        pltpu.VMEM_SHARED((NPAD,), jnp.float32),
    ],
    compiler_params=pltpu.CompilerParams(needs_layout_passes=False,
                                         use_tc_tiling_on_sc=False),
)
def _deg(dst_hbm, ew_hbm, out_hbm, idx_v, val_v, zero_v, acc):
    c = lax.axis_index("c")
    s = lax.axis_index("s")
    wid = c * NS + s

    def zfill(i, _):
        zero_v[pl.ds(i * 16, 16)] = jnp.zeros((16,), jnp.float32)
        return 0

    lax.fori_loop(0, RS // 16, zfill, 0)
    pltpu.sync_copy(zero_v, acc.at[pl.ds(s * RS, RS)])
    plsc.subcore_barrier()

    base = wid * NB * B

    def body(b, _):
        off = base + b * B
        pltpu.sync_copy(dst_hbm.at[pl.ds(off, B)], idx_v)
        pltpu.sync_copy(ew_hbm.at[pl.ds(off, B)], val_v)
        pltpu.sync_copy(val_v, acc.at[idx_v], add=True)
        return 0

    lax.fori_loop(0, NB, body, 0)
    plsc.subcore_barrier()
    pltpu.sync_copy(acc.at[pl.ds(s * RS, RS)], out_hbm.at[c, pl.ds(s * RS, RS)])


@functools.partial(
    pl.kernel,
    out_type=jax.ShapeDtypeStruct((NC, NPAD, D), jnp.float32),
    mesh=_mesh,
    scratch_types=(
        [
            pltpu.VMEM((NB * B,), jnp.int32),
            pltpu.VMEM((NB, B), jnp.int32),
            pltpu.VMEM((NB * B,), jnp.float32),
        ]
        + [pltpu.VMEM((B, D), jnp.float32) for _ in range(NBUF)]
        + [
            pltpu.VMEM((64, D), jnp.float32),
            pltpu.VMEM_SHARED((NPAD, D), jnp.float32),
        ]
        + [pltpu.SemaphoreType.DMA for _ in range(2 * NBUF)]
    ),
    compiler_params=pltpu.CompilerParams(needs_layout_passes=False,
                                         use_tc_tiling_on_sc=False),
)
def _agg(y_hbm, src_hbm, dst_hbm, ew_hbm, out_hbm, *refs):
    src_c, dst_c, ew_c = refs[0], refs[1], refs[2]
    rows = refs[3:3 + NBUF]
    zbuf, acc = refs[3 + NBUF], refs[4 + NBUF]
    gsem = refs[5 + NBUF:5 + 2 * NBUF]
    ssem = refs[5 + 2 * NBUF:5 + 3 * NBUF]

    c = lax.axis_index("c")
    s = lax.axis_index("s")
    wid = c * NS + s

    pltpu.sync_copy(src_hbm.at[wid], src_c)
    pltpu.sync_copy(dst_hbm.at[wid], dst_c)
    pltpu.sync_copy(ew_hbm.at[wid], ew_c)

    for r in range(64):
        for j in range(D // 16):
            zbuf[r, pl.ds(j * 16, 16)] = jnp.zeros((16,), jnp.float32)
    for t in range(RS // 64):
        pltpu.sync_copy(zbuf, acc.at[pl.ds(s * RS + t * 64, 64)])
    plsc.subcore_barrier()

    def _gather(b, k):
        return pltpu.async_copy(
            y_hbm.at[src_c.at[pl.ds(b * B, B)]], rows[k], gsem[k])

    def _scatter(b, k):
        return pltpu.async_copy(rows[k], acc.at[dst_c.at[b]], ssem[k],
                                add=True)

    def _gather_wait(b, k):
        pltpu.make_async_copy(
            y_hbm.at[src_c.at[pl.ds(b * B, B)]], rows[k], gsem[k]).wait()

    def _scatter_wait(b, k):
        pltpu.make_async_copy(rows[k], acc.at[dst_c.at[b]], ssem[k]).wait()

    for k in range(NBUF):
        _gather(k, k)

    def body(i, _):
        sbase = i * NBUF
        for k in range(NBUF):
            b = sbase + k
            _gather_wait(b, k)

            def gbody(g, _, k=k, b=b):
                base = b * B + g * 16
                for e16 in range(16):
                    w = plsc.load_gather(
                        ew_c, [jnp.full((16,), base + e16, jnp.int32)])
                    r = g * 16 + e16
                    for j in range(D // 16):
                        rows[k][r, pl.ds(j * 16, 16)] = (
                            rows[k][r, pl.ds(j * 16, 16)] * w)
                return 0

            lax.fori_loop(0, B // 16, gbody, 0)
            _scatter(b, k)
        for k in range(NBUF):
            b = sbase + k
            _scatter_wait(b, k)

            @pl.when(b + NBUF < NB)
            def _(k=k, b=b):
                _gather(b + NBUF, k)
        return 0

    lax.fori_loop(0, NB // NBUF, body, 0)
    plsc.subcore_barrier()
    pltpu.sync_copy(acc.at[pl.ds(s * RS, RS)], out_hbm.at[c, pl.ds(s * RS, RS)])


def _tc1_body(degT_ref, x_ref, W1_ref, dis_ref, y_ref):
    deg = 1.0 + degT_ref[:, 0:1] + degT_ref[:, 1:2]
    dis = jnp.where(deg > 0, lax.rsqrt(deg), 0.0)
    dis_ref[...] = dis
    y_ref[...] = dis * jnp.dot(x_ref[...], W1_ref[...],
                               preferred_element_type=jnp.float32,
                               precision=lax.Precision.DEFAULT)


_tc1 = pl.pallas_call(
    _tc1_body,
    grid=(GRID_M,),
    in_specs=[
        pl.BlockSpec((MBLK, 2), lambda i: (i, 0)),
        pl.BlockSpec((MBLK, D_IN), lambda i: (i, 0)),
        pl.BlockSpec((D_IN, D), lambda i: (0, 0)),
    ],
    out_specs=[
        pl.BlockSpec((MBLK, 1), lambda i: (i, 0)),
        pl.BlockSpec((MBLK, D), lambda i: (i, 0)),
    ],
    out_shape=[
        jax.ShapeDtypeStruct((NPAD, 1), jnp.float32),
        jax.ShapeDtypeStruct((NPAD, D), jnp.float32),
    ],
)


def _tclayer_body(s0_ref, s1_ref, yp_ref, dis_ref, b_ref, W_ref, y_ref):
    dis = dis_ref[...]
    s = s0_ref[...] + s1_ref[...] + yp_ref[...]
    h = jax.nn.relu(dis * s + b_ref[...])
    y_ref[...] = dis * jnp.dot(h, W_ref[...], preferred_element_type=jnp.float32,
                               precision=lax.Precision.DEFAULT)


_tclayer = pl.pallas_call(
    _tclayer_body,
    grid=(GRID_M,),
    in_specs=[
        pl.BlockSpec((MBLK, D), lambda i: (i, 0)),
        pl.BlockSpec((MBLK, D), lambda i: (i, 0)),
        pl.BlockSpec((MBLK, D), lambda i: (i, 0)),
        pl.BlockSpec((MBLK, 1), lambda i: (i, 0)),
        pl.BlockSpec((1, D), lambda i: (0, 0)),
        pl.BlockSpec((D, D), lambda i: (0, 0)),
    ],
    out_specs=pl.BlockSpec((MBLK, D), lambda i: (i, 0)),
    out_shape=jax.ShapeDtypeStruct((NPAD, D), jnp.float32),
)


def _tcfinal_body(s0_ref, s1_ref, yp_ref, dis_ref, b5_ref, fc1W_ref, fc1b_ref,
                  fc2W_ref, fc2b_ref, o_ref):
    dis = dis_ref[...]
    s = s0_ref[...] + s1_ref[...] + yp_ref[...]
    h = jax.nn.relu(dis * s + b5_ref[...])
    h = jax.nn.relu(jnp.dot(h, fc1W_ref[...], preferred_element_type=jnp.float32,
                               precision=lax.Precision.DEFAULT)
                    + fc1b_ref[...])
    o_ref[...] = jnp.dot(h, fc2W_ref[...], preferred_element_type=jnp.float32,
                               precision=lax.Precision.DEFAULT) \
        + fc2b_ref[...]


_tcfinal = pl.pallas_call(
    _tcfinal_body,
    grid=(GRID_M,),
    in_specs=[
        pl.BlockSpec((MBLK, D), lambda i: (i, 0)),
        pl.BlockSpec((MBLK, D), lambda i: (i, 0)),
        pl.BlockSpec((MBLK, D), lambda i: (i, 0)),
        pl.BlockSpec((MBLK, 1), lambda i: (i, 0)),
        pl.BlockSpec((1, D), lambda i: (0, 0)),
        pl.BlockSpec((D, D), lambda i: (0, 0)),
        pl.BlockSpec((1, D), lambda i: (0, 0)),
        pl.BlockSpec((D, 128), lambda i: (0, 0)),
        pl.BlockSpec((1, 128), lambda i: (0, 0)),
    ],
    out_specs=pl.BlockSpec((MBLK, 128), lambda i: (i, 0)),
    out_shape=jax.ShapeDtypeStruct((NPAD, 128), jnp.float32),
)


def kernel(x, edge_index, edge_attr, W1, b1, W2, b2, W3, b3, W4, b4, W5, b5,
           fc1_W, fc1_b, fc2_W, fc2_b):
    src = edge_index[0].astype(jnp.int32)
    dst = edge_index[1].astype(jnp.int32)
    ew = edge_attr.astype(jnp.float32)

    npad_rows = E_PAD - E
    padidx = N + (jnp.arange(npad_rows, dtype=jnp.int32) % (NPAD - N))
    srcp = jnp.concatenate([src, padidx])
    dstp = jnp.concatenate([dst, padidx])
    ewp = jnp.concatenate([ew, jnp.zeros((npad_rows,), jnp.float32)])
    xp = jnp.pad(x, ((0, NPAD - N), (0, 0)))

    src3 = srcp.reshape(NW, NB * B)
    dst3 = dstp.reshape(NW, NB, B)
    ew3 = ewp.reshape(NW, NB * B)

    deg01 = _deg(dstp, ewp)
    degT = deg01.T
    dis, y = _tc1(degT, xp, W1)

    bs = [b1.reshape(1, D), b2.reshape(1, D), b3.reshape(1, D),
          b4.reshape(1, D), b5.reshape(1, D)]
    Ws = [W2, W3, W4, W5]

    for li in range(4):
        sp = _agg(y, src3, dst3, ew3)
        y = _tclayer(sp[0], sp[1], y, dis, bs[li], Ws[li])
    sp = _agg(y, src3, dst3, ew3)

    fc2_Wp = jnp.pad(fc2_W, ((0, 0), (0, 128 - fc2_W.shape[1])))
    fc2_bp = jnp.pad(fc2_b.reshape(1, -1), ((0, 0), (0, 128 - fc2_b.shape[0])))
    out = _tcfinal(sp[0], sp[1], y, dis, bs[4], fc1_W,
                   fc1_b.reshape(1, D), fc2_Wp, fc2_bp)
    return out[:N, :1]


# speedup vs baseline: 18.5665x; 1.1494x over previous
import functools

import jax
import jax.numpy as jnp
from jax import lax
from jax.experimental import pallas as pl
from jax.experimental.pallas import tpu as pltpu
from jax.experimental.pallas import tpu_sc as plsc

N = 10000
E = 320000
D_IN = 128
D = 64

NC = 2
NS = 16
NW = NC * NS
B = 128
NB = 80
E_PAD = NW * NB * B
NBUF = 4
NPAD = 10240
RS = NPAD // NS
MBLK = 1024
GRID_M = NPAD // MBLK

_mesh = plsc.VectorSubcoreMesh(core_axis_name="c", subcore_axis_name="s")


@functools.partial(
    pl.kernel,
    out_type=jax.ShapeDtypeStruct((NC, NPAD), jnp.float32),
    mesh=_mesh,
    scratch_types=[
        pltpu.VMEM((B,), jnp.int32),
        pltpu.VMEM((B,), jnp.float32),
        pltpu.VMEM((RS,), jnp.float32),
        pltpu.VMEM_SHARED((NPAD,), jnp.float32),
    ],
    compiler_params=pltpu.CompilerParams(needs_layout_passes=False,
                                         use_tc_tiling_on_sc=False),
)
def _deg(dst_hbm, ew_hbm, out_hbm, idx_v, val_v, zero_v, acc):
    c = lax.axis_index("c")
    s = lax.axis_index("s")
    wid = c * NS + s

    def zfill(i, _):
        zero_v[pl.ds(i * 16, 16)] = jnp.zeros((16,), jnp.float32)
        return 0

    lax.fori_loop(0, RS // 16, zfill, 0)
    pltpu.sync_copy(zero_v, acc.at[pl.ds(s * RS, RS)])
    plsc.subcore_barrier()

    base = wid * NB * B

    def body(b, _):
        off = base + b * B
        pltpu.sync_copy(dst_hbm.at[pl.ds(off, B)], idx_v)
        pltpu.sync_copy(ew_hbm.at[pl.ds(off, B)], val_v)
        pltpu.sync_copy(val_v, acc.at[idx_v], add=True)
        return 0

    lax.fori_loop(0, NB, body, 0)
    plsc.subcore_barrier()
    pltpu.sync_copy(acc.at[pl.ds(s * RS, RS)], out_hbm.at[c, pl.ds(s * RS, RS)])


@functools.partial(
    pl.kernel,
    out_type=jax.ShapeDtypeStruct((NC, NPAD, D), jnp.float32),
    mesh=_mesh,
    scratch_types=(
        [
            pltpu.VMEM((NB * B,), jnp.int32),
            pltpu.VMEM((NB, B), jnp.int32),
            pltpu.VMEM((NB * B,), jnp.float32),
        ]
        + [pltpu.VMEM((B, D), jnp.float32) for _ in range(NBUF)]
        + [
            pltpu.VMEM((64, D), jnp.float32),
            pltpu.VMEM_SHARED((NPAD, D), jnp.float32),
        ]
        + [pltpu.SemaphoreType.DMA for _ in range(2 * NBUF)]
    ),
    compiler_params=pltpu.CompilerParams(needs_layout_passes=False,
                                         use_tc_tiling_on_sc=False),
)
def _agg(y_hbm, src_hbm, dst_hbm, ew_hbm, out_hbm, *refs):
    src_c, dst_c, ew_c = refs[0], refs[1], refs[2]
    rows = refs[3:3 + NBUF]
    zbuf, acc = refs[3 + NBUF], refs[4 + NBUF]
    gsem = refs[5 + NBUF:5 + 2 * NBUF]
    ssem = refs[5 + 2 * NBUF:5 + 3 * NBUF]

    c = lax.axis_index("c")
    s = lax.axis_index("s")
    wid = c * NS + s

    pltpu.sync_copy(src_hbm.at[wid], src_c)
    pltpu.sync_copy(dst_hbm.at[wid], dst_c)
    pltpu.sync_copy(ew_hbm.at[wid], ew_c)

    for r in range(64):
        for j in range(D // 16):
            zbuf[r, pl.ds(j * 16, 16)] = jnp.zeros((16,), jnp.float32)
    for t in range(RS // 64):
        pltpu.sync_copy(zbuf, acc.at[pl.ds(s * RS + t * 64, 64)])
    plsc.subcore_barrier()

    def _gather(b, k):
        return pltpu.async_copy(
            y_hbm.at[src_c.at[pl.ds(b * B, B)]], rows[k], gsem[k])

    def _scatter(b, k):
        return pltpu.async_copy(rows[k], acc.at[dst_c.at[b]], ssem[k],
                                add=True)

    def _gather_wait(b, k):
        pltpu.make_async_copy(
            y_hbm.at[src_c.at[pl.ds(b * B, B)]], rows[k], gsem[k]).wait()

    def _scatter_wait(b, k):
        pltpu.make_async_copy(rows[k], acc.at[dst_c.at[b]], ssem[k]).wait()

    for k in range(NBUF):
        _gather(k, k)

    def body(i, _):
        sbase = i * NBUF
        for k in range(NBUF):
            b = sbase + k
            _gather_wait(b, k)

            def gbody(g, _, k=k, b=b):
                base = b * B + g * 16
                for e16 in range(16):
                    w = plsc.load_gather(
                        ew_c, [jnp.full((16,), base + e16, jnp.int32)])
                    r = g * 16 + e16
                    for j in range(D // 16):
                        rows[k][r, pl.ds(j * 16, 16)] = (
                            rows[k][r, pl.ds(j * 16, 16)] * w)
                return 0

            lax.fori_loop(0, B // 16, gbody, 0)
            _scatter(b, k)
        for k in range(NBUF):
            b = sbase + k
            _scatter_wait(b, k)

            @pl.when(b + NBUF < NB)
            def _(k=k, b=b):
                _gather(b + NBUF, k)
        return 0

    lax.fori_loop(0, NB // NBUF, body, 0)
    plsc.subcore_barrier()
    pltpu.sync_copy(acc.at[pl.ds(s * RS, RS)], out_hbm.at[c, pl.ds(s * RS, RS)])


def _dis_of(degT):
    deg = 1.0 + degT[:, 0:1] + degT[:, 1:2]
    return jnp.where(deg > 0, lax.rsqrt(deg), 0.0)


def _tc1_body(degT_ref, x_ref, W1_ref, y_ref):
    y_ref[...] = _dis_of(degT_ref[...]) * jnp.dot(
        x_ref[...], W1_ref[...], preferred_element_type=jnp.float32,
        precision=lax.Precision.DEFAULT)


_tc1 = pl.pallas_call(
    _tc1_body,
    grid=(GRID_M,),
    in_specs=[
        pl.BlockSpec((MBLK, 2), lambda i: (i, 0)),
        pl.BlockSpec((MBLK, D_IN), lambda i: (i, 0)),
        pl.BlockSpec((D_IN, D), lambda i: (0, 0)),
    ],
    out_specs=pl.BlockSpec((MBLK, D), lambda i: (i, 0)),
    out_shape=jax.ShapeDtypeStruct((NPAD, D), jnp.float32),
)


def _tclayer_body(sp_ref, yp_ref, degT_ref, b_ref, W_ref, y_ref):
    dis = _dis_of(degT_ref[...])
    s = sp_ref[0] + sp_ref[1] + yp_ref[...]
    h = jax.nn.relu(dis * s + b_ref[...])
    y_ref[...] = dis * jnp.dot(h, W_ref[...], preferred_element_type=jnp.float32,
                               precision=lax.Precision.DEFAULT)


_tclayer = pl.pallas_call(
    _tclayer_body,
    grid=(GRID_M,),
    in_specs=[
        pl.BlockSpec((NC, MBLK, D), lambda i: (0, i, 0)),
        pl.BlockSpec((MBLK, D), lambda i: (i, 0)),
        pl.BlockSpec((MBLK, 2), lambda i: (i, 0)),
        pl.BlockSpec((1, D), lambda i: (0, 0)),
        pl.BlockSpec((D, D), lambda i: (0, 0)),
    ],
    out_specs=pl.BlockSpec((MBLK, D), lambda i: (i, 0)),
    out_shape=jax.ShapeDtypeStruct((NPAD, D), jnp.float32),
)


def _tcfinal_body(sp_ref, yp_ref, degT_ref, b5_ref, fc1W_ref, fc1b_ref,
                  fc2W_ref, fc2b_ref, o_ref):
    dis = _dis_of(degT_ref[...])
    s = sp_ref[0] + sp_ref[1] + yp_ref[...]
    h = jax.nn.relu(dis * s + b5_ref[...])
    h = jax.nn.relu(jnp.dot(h, fc1W_ref[...], preferred_element_type=jnp.float32,
                            precision=lax.Precision.DEFAULT) + fc1b_ref[...])
    o_ref[...] = jnp.dot(h, fc2W_ref[...], preferred_element_type=jnp.float32,
                         precision=lax.Precision.DEFAULT) + fc2b_ref[...]


_tcfinal = pl.pallas_call(
    _tcfinal_body,
    grid=(GRID_M,),
    in_specs=[
        pl.BlockSpec((NC, MBLK, D), lambda i: (0, i, 0)),
        pl.BlockSpec((MBLK, D), lambda i: (i, 0)),
        pl.BlockSpec((MBLK, 2), lambda i: (i, 0)),
        pl.BlockSpec((1, D), lambda i: (0, 0)),
        pl.BlockSpec((D, D), lambda i: (0, 0)),
        pl.BlockSpec((1, D), lambda i: (0, 0)),
        pl.BlockSpec((D, 128), lambda i: (0, 0)),
        pl.BlockSpec((1, 128), lambda i: (0, 0)),
    ],
    out_specs=pl.BlockSpec((MBLK, 128), lambda i: (i, 0)),
    out_shape=jax.ShapeDtypeStruct((NPAD, 128), jnp.float32),
)


def kernel(x, edge_index, edge_attr, W1, b1, W2, b2, W3, b3, W4, b4, W5, b5,
           fc1_W, fc1_b, fc2_W, fc2_b):
    src = edge_index[0].astype(jnp.int32)
    dst = edge_index[1].astype(jnp.int32)
    ew = edge_attr.astype(jnp.float32)

    npad_rows = E_PAD - E
    padidx = N + (jnp.arange(npad_rows, dtype=jnp.int32) % (NPAD - N))
    srcp = jnp.concatenate([src, padidx])
    dstp = jnp.concatenate([dst, padidx])
    ewp = jnp.concatenate([ew, jnp.zeros((npad_rows,), jnp.float32)])
    xp = jnp.pad(x, ((0, NPAD - N), (0, 0)))

    src3 = srcp.reshape(NW, NB * B)
    dst3 = dstp.reshape(NW, NB, B)
    ew3 = ewp.reshape(NW, NB * B)

    deg01 = _deg(dstp, ewp)
    degT = deg01.T
    y = _tc1(degT, xp, W1)

    bs = [b1.reshape(1, D), b2.reshape(1, D), b3.reshape(1, D),
          b4.reshape(1, D), b5.reshape(1, D)]
    Ws = [W2, W3, W4, W5]

    for li in range(4):
        sp = _agg(y, src3, dst3, ew3)
        y = _tclayer(sp, y, degT, bs[li], Ws[li])
    sp = _agg(y, src3, dst3, ew3)

    fc2_Wp = jnp.pad(fc2_W, ((0, 0), (0, 128 - fc2_W.shape[1])))
    fc2_bp = jnp.pad(fc2_b.reshape(1, -1), ((0, 0), (0, 128 - fc2_b.shape[0])))
    out = _tcfinal(sp, y, degT, bs[4], fc1_W,
                   fc1_b.reshape(1, D), fc2_Wp, fc2_bp)
    return out[:N, :1]


# speedup vs baseline: 20.3411x; 1.0956x over previous
import functools

import jax
import jax.numpy as jnp
from jax import lax
from jax.experimental import pallas as pl
from jax.experimental.pallas import tpu as pltpu
from jax.experimental.pallas import tpu_sc as plsc

N = 10000
E = 320000
D_IN = 128
D = 64

NC = 2
NS = 16
NW = NC * NS
B = 128
NB = 80
E_PAD = NW * NB * B
NBUF = 4
NPAD = 10240
RS = NPAD // NS
MBLK = 1024
GRID_M = NPAD // MBLK

_mesh = plsc.VectorSubcoreMesh(core_axis_name="c", subcore_axis_name="s")


@functools.partial(
    pl.kernel,
    out_type=jax.ShapeDtypeStruct((NC, NPAD), jnp.float32),
    mesh=_mesh,
    scratch_types=[
        pltpu.VMEM((NB, B), jnp.int32),
        pltpu.VMEM((NB * B,), jnp.float32),
        pltpu.VMEM((RS,), jnp.float32),
        pltpu.VMEM_SHARED((NPAD,), jnp.float32),
        pltpu.SemaphoreType.DMA,
    ],
    compiler_params=pltpu.CompilerParams(needs_layout_passes=False,
                                         use_tc_tiling_on_sc=False),
)
def _deg(dst_hbm, ew_hbm, out_hbm, dst_c, ew_c, zero_v, acc, sem):
    c = lax.axis_index("c")
    s = lax.axis_index("s")
    wid = c * NS + s

    pltpu.sync_copy(dst_hbm.at[wid], dst_c)
    pltpu.sync_copy(ew_hbm.at[wid], ew_c)

    def zfill(i, _):
        zero_v[pl.ds(i * 16, 16)] = jnp.zeros((16,), jnp.float32)
        return 0

    lax.fori_loop(0, RS // 16, zfill, 0)
    pltpu.sync_copy(zero_v, acc.at[pl.ds(s * RS, RS)])
    plsc.subcore_barrier()

    def fire(i, _):
        for k in range(8):
            b = i * 8 + k
            pltpu.async_copy(ew_c.at[pl.ds(b * B, B)], acc.at[dst_c.at[b]],
                             sem, add=True)
        return 0

    lax.fori_loop(0, NB // 8, fire, 0)

    def drain(i, _):
        for k in range(8):
            b = i * 8 + k
            pltpu.make_async_copy(ew_c.at[pl.ds(b * B, B)],
                                  acc.at[dst_c.at[b]], sem).wait()
        return 0

    lax.fori_loop(0, NB // 8, drain, 0)
    plsc.subcore_barrier()
    pltpu.sync_copy(acc.at[pl.ds(s * RS, RS)], out_hbm.at[c, pl.ds(s * RS, RS)])


@functools.partial(
    pl.kernel,
    out_type=jax.ShapeDtypeStruct((NC, NPAD, D), jnp.float32),
    mesh=_mesh,
    scratch_types=(
        [
            pltpu.VMEM((NB * B,), jnp.int32),
            pltpu.VMEM((NB, B), jnp.int32),
            pltpu.VMEM((NB * B,), jnp.float32),
        ]
        + [pltpu.VMEM((B, D), jnp.float32) for _ in range(NBUF)]
        + [
            pltpu.VMEM((64, D), jnp.float32),
            pltpu.VMEM_SHARED((NPAD, D), jnp.float32),
        ]
        + [pltpu.SemaphoreType.DMA for _ in range(2 * NBUF)]
    ),
    compiler_params=pltpu.CompilerParams(needs_layout_passes=False,
                                         use_tc_tiling_on_sc=False),
)
def _agg(y_hbm, src_hbm, dst_hbm, ew_hbm, out_hbm, *refs):
    src_c, dst_c, ew_c = refs[0], refs[1], refs[2]
    rows = refs[3:3 + NBUF]
    zbuf, acc = refs[3 + NBUF], refs[4 + NBUF]
    gsem = refs[5 + NBUF:5 + 2 * NBUF]
    ssem = refs[5 + 2 * NBUF:5 + 3 * NBUF]

    c = lax.axis_index("c")
    s = lax.axis_index("s")
    wid = c * NS + s

    pltpu.sync_copy(src_hbm.at[wid], src_c)
    pltpu.sync_copy(dst_hbm.at[wid], dst_c)
    pltpu.sync_copy(ew_hbm.at[wid], ew_c)

    for r in range(64):
        for j in range(D // 16):
            zbuf[r, pl.ds(j * 16, 16)] = jnp.zeros((16,), jnp.float32)
    for t in range(RS // 64):
        pltpu.sync_copy(zbuf, acc.at[pl.ds(s * RS + t * 64, 64)])
    plsc.subcore_barrier()

    def _gather(b, k):
        return pltpu.async_copy(
            y_hbm.at[src_c.at[pl.ds(b * B, B)]], rows[k], gsem[k])

    def _scatter(b, k):
        return pltpu.async_copy(rows[k], acc.at[dst_c.at[b]], ssem[k],
                                add=True)

    def _gather_wait(b, k):
        pltpu.make_async_copy(
            y_hbm.at[src_c.at[pl.ds(b * B, B)]], rows[k], gsem[k]).wait()

    def _scatter_wait(b, k):
        pltpu.make_async_copy(rows[k], acc.at[dst_c.at[b]], ssem[k]).wait()

    for k in range(NBUF):
        _gather(k, k)

    def body(i, _):
        sbase = i * NBUF
        for k in range(NBUF):
            b = sbase + k
            _gather_wait(b, k)

            def gbody(g, _, k=k, b=b):
                base = b * B + g * 16
                for e16 in range(16):
                    w = plsc.load_gather(
                        ew_c, [jnp.full((16,), base + e16, jnp.int32)])
                    r = g * 16 + e16
                    for j in range(D // 16):
                        rows[k][r, pl.ds(j * 16, 16)] = (
                            rows[k][r, pl.ds(j * 16, 16)] * w)
                return 0

            lax.fori_loop(0, B // 16, gbody, 0)
            _scatter(b, k)
        for k in range(NBUF):
            b = sbase + k
            _scatter_wait(b, k)

            @pl.when(b + NBUF < NB)
            def _(k=k, b=b):
                _gather(b + NBUF, k)
        return 0

    lax.fori_loop(0, NB // NBUF, body, 0)
    plsc.subcore_barrier()
    pltpu.sync_copy(acc.at[pl.ds(s * RS, RS)], out_hbm.at[c, pl.ds(s * RS, RS)])


def _dis_of(degT):
    deg = 1.0 + degT[:, 0:1] + degT[:, 1:2]
    return jnp.where(deg > 0, lax.rsqrt(deg), 0.0)


def _tc1_body(degT_ref, x_ref, W1_ref, y_ref):
    y_ref[...] = _dis_of(degT_ref[...]) * jnp.dot(
        x_ref[...], W1_ref[...], preferred_element_type=jnp.float32,
        precision=lax.Precision.DEFAULT)


_tc1 = pl.pallas_call(
    _tc1_body,
    grid=(GRID_M,),
    in_specs=[
        pl.BlockSpec((MBLK, 2), lambda i: (i, 0)),
        pl.BlockSpec((MBLK, D_IN), lambda i: (i, 0)),
        pl.BlockSpec((D_IN, D), lambda i: (0, 0)),
    ],
    out_specs=pl.BlockSpec((MBLK, D), lambda i: (i, 0)),
    out_shape=jax.ShapeDtypeStruct((NPAD, D), jnp.float32),
)


def _tclayer_body(sp_ref, yp_ref, degT_ref, b_ref, W_ref, y_ref):
    dis = _dis_of(degT_ref[...])
    s = sp_ref[0] + sp_ref[1] + yp_ref[...]
    h = jax.nn.relu(dis * s + b_ref[...])
    y_ref[...] = dis * jnp.dot(h, W_ref[...], preferred_element_type=jnp.float32,
                               precision=lax.Precision.DEFAULT)


_tclayer = pl.pallas_call(
    _tclayer_body,
    grid=(GRID_M,),
    in_specs=[
        pl.BlockSpec((NC, MBLK, D), lambda i: (0, i, 0)),
        pl.BlockSpec((MBLK, D), lambda i: (i, 0)),
        pl.BlockSpec((MBLK, 2), lambda i: (i, 0)),
        pl.BlockSpec((1, D), lambda i: (0, 0)),
        pl.BlockSpec((D, D), lambda i: (0, 0)),
    ],
    out_specs=pl.BlockSpec((MBLK, D), lambda i: (i, 0)),
    out_shape=jax.ShapeDtypeStruct((NPAD, D), jnp.float32),
)


def _tcfinal_body(sp_ref, yp_ref, degT_ref, b5_ref, fc1W_ref, fc1b_ref,
                  fc2W_ref, fc2b_ref, o_ref):
    dis = _dis_of(degT_ref[...])
    s = sp_ref[0] + sp_ref[1] + yp_ref[...]
    h = jax.nn.relu(dis * s + b5_ref[...])
    h = jax.nn.relu(jnp.dot(h, fc1W_ref[...], preferred_element_type=jnp.float32,
                            precision=lax.Precision.DEFAULT) + fc1b_ref[...])
    o_ref[...] = jnp.dot(h, fc2W_ref[...], preferred_element_type=jnp.float32,
                         precision=lax.Precision.DEFAULT) + fc2b_ref[...]


_tcfinal = pl.pallas_call(
    _tcfinal_body,
    grid=(GRID_M,),
    in_specs=[
        pl.BlockSpec((NC, MBLK, D), lambda i: (0, i, 0)),
        pl.BlockSpec((MBLK, D), lambda i: (i, 0)),
        pl.BlockSpec((MBLK, 2), lambda i: (i, 0)),
        pl.BlockSpec((1, D), lambda i: (0, 0)),
        pl.BlockSpec((D, D), lambda i: (0, 0)),
        pl.BlockSpec((1, D), lambda i: (0, 0)),
        pl.BlockSpec((D, 128), lambda i: (0, 0)),
        pl.BlockSpec((1, 128), lambda i: (0, 0)),
    ],
    out_specs=pl.BlockSpec((MBLK, 128), lambda i: (i, 0)),
    out_shape=jax.ShapeDtypeStruct((NPAD, 128), jnp.float32),
)


def kernel(x, edge_index, edge_attr, W1, b1, W2, b2, W3, b3, W4, b4, W5, b5,
           fc1_W, fc1_b, fc2_W, fc2_b):
    src = edge_index[0].astype(jnp.int32)
    dst = edge_index[1].astype(jnp.int32)
    ew = edge_attr.astype(jnp.float32)

    npad_rows = E_PAD - E
    padidx = N + (jnp.arange(npad_rows, dtype=jnp.int32) % (NPAD - N))
    srcp = jnp.concatenate([src, padidx])
    dstp = jnp.concatenate([dst, padidx])
    ewp = jnp.concatenate([ew, jnp.zeros((npad_rows,), jnp.float32)])
    xp = jnp.pad(x, ((0, NPAD - N), (0, 0)))

    src3 = srcp.reshape(NW, NB * B)
    dst3 = dstp.reshape(NW, NB, B)
    ew3 = ewp.reshape(NW, NB * B)

    deg01 = _deg(dst3, ew3)
    degT = deg01.T
    y = _tc1(degT, xp, W1)

    bs = [b1.reshape(1, D), b2.reshape(1, D), b3.reshape(1, D),
          b4.reshape(1, D), b5.reshape(1, D)]
    Ws = [W2, W3, W4, W5]

    for li in range(4):
        sp = _agg(y, src3, dst3, ew3)
        y = _tclayer(sp, y, degT, bs[li], Ws[li])
    sp = _agg(y, src3, dst3, ew3)

    fc2_Wp = jnp.pad(fc2_W, ((0, 0), (0, 128 - fc2_W.shape[1])))
    fc2_bp = jnp.pad(fc2_b.reshape(1, -1), ((0, 0), (0, 128 - fc2_b.shape[0])))
    out = _tcfinal(sp, y, degT, bs[4], fc1_W,
                   fc1_b.reshape(1, D), fc2_Wp, fc2_bp)
    return out[:N, :1]


# speedup vs baseline: 28.4404x; 1.3982x over previous
import functools

import jax
import jax.numpy as jnp
from jax import lax
from jax.experimental import pallas as pl
from jax.experimental.pallas import tpu as pltpu
from jax.experimental.pallas import tpu_sc as plsc

N = 10000
E = 320000
D_IN = 128
D = 64

NC = 2
NS = 16
NW = NC * NS
B = 128
NB = 80
E_PAD = NW * NB * B
NBUF = 4
NPAD = 10240
RS = NPAD // NS
MBLK = 1024
GRID_M = NPAD // MBLK

_mesh = plsc.VectorSubcoreMesh(core_axis_name="c", subcore_axis_name="s")


@functools.partial(
    pl.kernel,
    out_type=jax.ShapeDtypeStruct((NC, NPAD), jnp.float32),
    mesh=_mesh,
    scratch_types=[
        pltpu.VMEM((NB, B), jnp.int32),
        pltpu.VMEM((NB * B,), jnp.float32),
        pltpu.VMEM((RS,), jnp.float32),
        pltpu.VMEM_SHARED((NPAD,), jnp.float32),
        pltpu.SemaphoreType.DMA,
    ],
    compiler_params=pltpu.CompilerParams(needs_layout_passes=False,
                                         use_tc_tiling_on_sc=False),
)
def _deg(dst_hbm, ew_hbm, out_hbm, dst_c, ew_c, zero_v, acc, sem):
    c = lax.axis_index("c")
    s = lax.axis_index("s")
    wid = c * NS + s

    pltpu.sync_copy(dst_hbm.at[wid], dst_c)
    pltpu.sync_copy(ew_hbm.at[wid], ew_c)

    def zfill(i, _):
        zero_v[pl.ds(i * 16, 16)] = jnp.zeros((16,), jnp.float32)
        return 0

    lax.fori_loop(0, RS // 16, zfill, 0)
    pltpu.sync_copy(zero_v, acc.at[pl.ds(s * RS, RS)])
    plsc.subcore_barrier()

    def fire(i, _):
        for k in range(8):
            b = i * 8 + k
            pltpu.async_copy(ew_c.at[pl.ds(b * B, B)], acc.at[dst_c.at[b]],
                             sem, add=True)
        return 0

    lax.fori_loop(0, NB // 8, fire, 0)

    def drain(i, _):
        for k in range(8):
            b = i * 8 + k
            pltpu.make_async_copy(ew_c.at[pl.ds(b * B, B)],
                                  acc.at[dst_c.at[b]], sem).wait()
        return 0

    lax.fori_loop(0, NB // 8, drain, 0)
    plsc.subcore_barrier()
    pltpu.sync_copy(acc.at[pl.ds(s * RS, RS)], out_hbm.at[c, pl.ds(s * RS, RS)])


@functools.partial(
    pl.kernel,
    out_type=jax.ShapeDtypeStruct((NC, NPAD, D), jnp.float32),
    mesh=_mesh,
    scratch_types=(
        [
            pltpu.VMEM((NB * B,), jnp.int32),
            pltpu.VMEM((NB, B), jnp.int32),
            pltpu.VMEM((NB * B,), jnp.float32),
        ]
        + [pltpu.VMEM((B, D), jnp.float32) for _ in range(NBUF)]
        + [
            pltpu.VMEM((64, D), jnp.float32),
            pltpu.VMEM_SHARED((NPAD, D), jnp.float32),
        ]
        + [pltpu.SemaphoreType.DMA for _ in range(2 * NBUF)]
    ),
    compiler_params=pltpu.CompilerParams(needs_layout_passes=False,
                                         use_tc_tiling_on_sc=False),
)
def _agg(y_hbm, src_hbm, dst_hbm, ew_hbm, out_hbm, *refs):
    src_c, dst_c, ew_c = refs[0], refs[1], refs[2]
    rows = refs[3:3 + NBUF]
    zbuf, acc = refs[3 + NBUF], refs[4 + NBUF]
    gsem = refs[5 + NBUF:5 + 2 * NBUF]
    ssem = refs[5 + 2 * NBUF:5 + 3 * NBUF]

    c = lax.axis_index("c")
    s = lax.axis_index("s")
    wid = c * NS + s

    pltpu.sync_copy(src_hbm.at[wid], src_c)
    pltpu.sync_copy(dst_hbm.at[wid], dst_c)
    pltpu.sync_copy(ew_hbm.at[wid], ew_c)

    for r in range(64):
        for j in range(D // 16):
            zbuf[r, pl.ds(j * 16, 16)] = jnp.zeros((16,), jnp.float32)
    for t in range(RS // 64):
        pltpu.sync_copy(zbuf, acc.at[pl.ds(s * RS + t * 64, 64)])
    plsc.subcore_barrier()

    def _gather(b, k):
        return pltpu.async_copy(
            y_hbm.at[src_c.at[pl.ds(b * B, B)]], rows[k], gsem[k])

    def _scatter(b, k):
        return pltpu.async_copy(rows[k], acc.at[dst_c.at[b]], ssem[k],
                                add=True)

    def _gather_wait(b, k):
        pltpu.make_async_copy(
            y_hbm.at[src_c.at[pl.ds(b * B, B)]], rows[k], gsem[k]).wait()

    def _scatter_wait(b, k):
        pltpu.make_async_copy(rows[k], acc.at[dst_c.at[b]], ssem[k]).wait()

    for k in range(NBUF):
        _gather(k, k)

    def body(i, _):
        sbase = i * NBUF
        for k in range(NBUF):
            b = sbase + k
            _gather_wait(b, k)

            def gbody(g, _, k=k, b=b):
                for h in range(2):
                    wvec = ew_c[pl.ds(b * B + g * 32 + h * 16, 16)]
                    for e16 in range(16):
                        w = jnp.full((16,), wvec[e16])
                        r = g * 32 + h * 16 + e16
                        for j in range(D // 16):
                            rows[k][r, pl.ds(j * 16, 16)] = (
                                rows[k][r, pl.ds(j * 16, 16)] * w)
                return 0

            lax.fori_loop(0, B // 32, gbody, 0)
            _scatter(b, k)
        for k in range(NBUF):
            b = sbase + k
            _scatter_wait(b, k)

            @pl.when(b + NBUF < NB)
            def _(k=k, b=b):
                _gather(b + NBUF, k)
        return 0

    lax.fori_loop(0, NB // NBUF, body, 0)
    plsc.subcore_barrier()
    pltpu.sync_copy(acc.at[pl.ds(s * RS, RS)], out_hbm.at[c, pl.ds(s * RS, RS)])


def _dis_of(degT):
    deg = 1.0 + degT[:, 0:1] + degT[:, 1:2]
    return jnp.where(deg > 0, lax.rsqrt(deg), 0.0)


def _tc1_body(degT_ref, x_ref, W1_ref, y_ref):
    y_ref[...] = _dis_of(degT_ref[...]) * jnp.dot(
        x_ref[...], W1_ref[...], preferred_element_type=jnp.float32,
        precision=lax.Precision.DEFAULT)


_tc1 = pl.pallas_call(
    _tc1_body,
    grid=(GRID_M,),
    in_specs=[
        pl.BlockSpec((MBLK, 2), lambda i: (i, 0)),
        pl.BlockSpec((MBLK, D_IN), lambda i: (i, 0)),
        pl.BlockSpec((D_IN, D), lambda i: (0, 0)),
    ],
    out_specs=pl.BlockSpec((MBLK, D), lambda i: (i, 0)),
    out_shape=jax.ShapeDtypeStruct((NPAD, D), jnp.float32),
)


def _tclayer_body(sp_ref, yp_ref, degT_ref, b_ref, W_ref, y_ref):
    dis = _dis_of(degT_ref[...])
    s = sp_ref[0] + sp_ref[1] + yp_ref[...]
    h = jax.nn.relu(dis * s + b_ref[...])
    y_ref[...] = dis * jnp.dot(h, W_ref[...], preferred_element_type=jnp.float32,
                               precision=lax.Precision.DEFAULT)


_tclayer = pl.pallas_call(
    _tclayer_body,
    grid=(GRID_M,),
    in_specs=[
        pl.BlockSpec((NC, MBLK, D), lambda i: (0, i, 0)),
        pl.BlockSpec((MBLK, D), lambda i: (i, 0)),
        pl.BlockSpec((MBLK, 2), lambda i: (i, 0)),
        pl.BlockSpec((1, D), lambda i: (0, 0)),
        pl.BlockSpec((D, D), lambda i: (0, 0)),
    ],
    out_specs=pl.BlockSpec((MBLK, D), lambda i: (i, 0)),
    out_shape=jax.ShapeDtypeStruct((NPAD, D), jnp.float32),
)


def _tcfinal_body(sp_ref, yp_ref, degT_ref, b5_ref, fc1W_ref, fc1b_ref,
                  fc2W_ref, fc2b_ref, o_ref):
    dis = _dis_of(degT_ref[...])
    s = sp_ref[0] + sp_ref[1] + yp_ref[...]
    h = jax.nn.relu(dis * s + b5_ref[...])
    h = jax.nn.relu(jnp.dot(h, fc1W_ref[...], preferred_element_type=jnp.float32,
                            precision=lax.Precision.DEFAULT) + fc1b_ref[...])
    o_ref[...] = jnp.dot(h, fc2W_ref[...], preferred_element_type=jnp.float32,
                         precision=lax.Precision.DEFAULT) + fc2b_ref[...]


_tcfinal = pl.pallas_call(
    _tcfinal_body,
    grid=(GRID_M,),
    in_specs=[
        pl.BlockSpec((NC, MBLK, D), lambda i: (0, i, 0)),
        pl.BlockSpec((MBLK, D), lambda i: (i, 0)),
        pl.BlockSpec((MBLK, 2), lambda i: (i, 0)),
        pl.BlockSpec((1, D), lambda i: (0, 0)),
        pl.BlockSpec((D, D), lambda i: (0, 0)),
        pl.BlockSpec((1, D), lambda i: (0, 0)),
        pl.BlockSpec((D, 128), lambda i: (0, 0)),
        pl.BlockSpec((1, 128), lambda i: (0, 0)),
    ],
    out_specs=pl.BlockSpec((MBLK, 128), lambda i: (i, 0)),
    out_shape=jax.ShapeDtypeStruct((NPAD, 128), jnp.float32),
)


def kernel(x, edge_index, edge_attr, W1, b1, W2, b2, W3, b3, W4, b4, W5, b5,
           fc1_W, fc1_b, fc2_W, fc2_b):
    src = edge_index[0].astype(jnp.int32)
    dst = edge_index[1].astype(jnp.int32)
    ew = edge_attr.astype(jnp.float32)

    npad_rows = E_PAD - E
    padidx = N + (jnp.arange(npad_rows, dtype=jnp.int32) % (NPAD - N))
    srcp = jnp.concatenate([src, padidx])
    dstp = jnp.concatenate([dst, padidx])
    ewp = jnp.concatenate([ew, jnp.zeros((npad_rows,), jnp.float32)])
    xp = jnp.pad(x, ((0, NPAD - N), (0, 0)))

    src3 = srcp.reshape(NW, NB * B)
    dst3 = dstp.reshape(NW, NB, B)
    ew3 = ewp.reshape(NW, NB * B)

    deg01 = _deg(dst3, ew3)
    degT = deg01.T
    y = _tc1(degT, xp, W1)

    bs = [b1.reshape(1, D), b2.reshape(1, D), b3.reshape(1, D),
          b4.reshape(1, D), b5.reshape(1, D)]
    Ws = [W2, W3, W4, W5]

    for li in range(4):
        sp = _agg(y, src3, dst3, ew3)
        y = _tclayer(sp, y, degT, bs[li], Ws[li])
    sp = _agg(y, src3, dst3, ew3)

    fc2_Wp = jnp.pad(fc2_W, ((0, 0), (0, 128 - fc2_W.shape[1])))
    fc2_bp = jnp.pad(fc2_b.reshape(1, -1), ((0, 0), (0, 128 - fc2_b.shape[0])))
    out = _tcfinal(sp, y, degT, bs[4], fc1_W,
                   fc1_b.reshape(1, D), fc2_Wp, fc2_bp)
    return out[:N, :1]


# speedup vs baseline: 28.6387x; 1.0070x over previous
import functools

import jax
import jax.numpy as jnp
from jax import lax
from jax.experimental import pallas as pl
from jax.experimental.pallas import tpu as pltpu
from jax.experimental.pallas import tpu_sc as plsc

N = 10000
E = 320000
D_IN = 128
D = 64

NC = 2
NS = 16
NW = NC * NS
B = 128
NB = 80
E_PAD = NW * NB * B
NBUF = 5
NPAD = 10240
RS = NPAD // NS
MBLK = 1024
GRID_M = NPAD // MBLK

_mesh = plsc.VectorSubcoreMesh(core_axis_name="c", subcore_axis_name="s")


@functools.partial(
    pl.kernel,
    out_type=jax.ShapeDtypeStruct((NC, NPAD), jnp.float32),
    mesh=_mesh,
    scratch_types=[
        pltpu.VMEM((NB, B), jnp.int32),
        pltpu.VMEM((NB * B,), jnp.float32),
        pltpu.VMEM((RS,), jnp.float32),
        pltpu.VMEM_SHARED((NPAD,), jnp.float32),
        pltpu.SemaphoreType.DMA,
    ],
    compiler_params=pltpu.CompilerParams(needs_layout_passes=False,
                                         use_tc_tiling_on_sc=False),
)
def _deg(dst_hbm, ew_hbm, out_hbm, dst_c, ew_c, zero_v, acc, sem):
    c = lax.axis_index("c")
    s = lax.axis_index("s")
    wid = c * NS + s

    pltpu.sync_copy(dst_hbm.at[wid], dst_c)
    pltpu.sync_copy(ew_hbm.at[wid], ew_c)

    def zfill(i, _):
        zero_v[pl.ds(i * 16, 16)] = jnp.zeros((16,), jnp.float32)
        return 0

    lax.fori_loop(0, RS // 16, zfill, 0)
    pltpu.sync_copy(zero_v, acc.at[pl.ds(s * RS, RS)])
    plsc.subcore_barrier()

    def fire(i, _):
        for k in range(8):
            b = i * 8 + k
            pltpu.async_copy(ew_c.at[pl.ds(b * B, B)], acc.at[dst_c.at[b]],
                             sem, add=True)
        return 0

    lax.fori_loop(0, NB // 8, fire, 0)

    def drain(i, _):
        for k in range(8):
            b = i * 8 + k
            pltpu.make_async_copy(ew_c.at[pl.ds(b * B, B)],
                                  acc.at[dst_c.at[b]], sem).wait()
        return 0

    lax.fori_loop(0, NB // 8, drain, 0)
    plsc.subcore_barrier()
    pltpu.sync_copy(acc.at[pl.ds(s * RS, RS)], out_hbm.at[c, pl.ds(s * RS, RS)])


@functools.partial(
    pl.kernel,
    out_type=jax.ShapeDtypeStruct((NC, NPAD, D), jnp.float32),
    mesh=_mesh,
    scratch_types=(
        [
            pltpu.VMEM((NB * B,), jnp.int32),
            pltpu.VMEM((NB, B), jnp.int32),
            pltpu.VMEM((NB * B,), jnp.float32),
        ]
        + [pltpu.VMEM((B, D), jnp.float32) for _ in range(NBUF)]
        + [
            pltpu.VMEM((64, D), jnp.float32),
            pltpu.VMEM_SHARED((NPAD, D), jnp.float32),
        ]
        + [pltpu.SemaphoreType.DMA for _ in range(2 * NBUF)]
    ),
    compiler_params=pltpu.CompilerParams(needs_layout_passes=False,
                                         use_tc_tiling_on_sc=False),
)
def _agg(y_hbm, src_hbm, dst_hbm, ew_hbm, out_hbm, *refs):
    src_c, dst_c, ew_c = refs[0], refs[1], refs[2]
    rows = refs[3:3 + NBUF]
    zbuf, acc = refs[3 + NBUF], refs[4 + NBUF]
    gsem = refs[5 + NBUF:5 + 2 * NBUF]
    ssem = refs[5 + 2 * NBUF:5 + 3 * NBUF]

    c = lax.axis_index("c")
    s = lax.axis_index("s")
    wid = c * NS + s

    pltpu.sync_copy(src_hbm.at[wid], src_c)
    pltpu.sync_copy(dst_hbm.at[wid], dst_c)
    pltpu.sync_copy(ew_hbm.at[wid], ew_c)

    for r in range(64):
        for j in range(D // 16):
            zbuf[r, pl.ds(j * 16, 16)] = jnp.zeros((16,), jnp.float32)
    for t in range(RS // 64):
        pltpu.sync_copy(zbuf, acc.at[pl.ds(s * RS + t * 64, 64)])
    plsc.subcore_barrier()

    def _gather(b, k):
        return pltpu.async_copy(
            y_hbm.at[src_c.at[pl.ds(b * B, B)]], rows[k], gsem[k])

    def _scatter(b, k):
        return pltpu.async_copy(rows[k], acc.at[dst_c.at[b]], ssem[k],
                                add=True)

    def _gather_wait(b, k):
        pltpu.make_async_copy(
            y_hbm.at[src_c.at[pl.ds(b * B, B)]], rows[k], gsem[k]).wait()

    def _scatter_wait(b, k):
        pltpu.make_async_copy(rows[k], acc.at[dst_c.at[b]], ssem[k]).wait()

    for k in range(NBUF):
        _gather(k, k)

    def body(i, _):
        sbase = i * NBUF
        for k in range(NBUF):
            b = sbase + k
            _gather_wait(b, k)

            def gbody(g, _, k=k, b=b):
                for h in range(2):
                    wvec = ew_c[pl.ds(b * B + g * 32 + h * 16, 16)]
                    for e16 in range(16):
                        w = jnp.full((16,), wvec[e16])
                        r = g * 32 + h * 16 + e16
                        for j in range(D // 16):
                            rows[k][r, pl.ds(j * 16, 16)] = (
                                rows[k][r, pl.ds(j * 16, 16)] * w)
                return 0

            lax.fori_loop(0, B // 32, gbody, 0)
            _scatter(b, k)
        for k in range(NBUF):
            b = sbase + k
            _scatter_wait(b, k)

            @pl.when(b + NBUF < NB)
            def _(k=k, b=b):
                _gather(b + NBUF, k)
        return 0

    lax.fori_loop(0, NB // NBUF, body, 0)
    plsc.subcore_barrier()
    pltpu.sync_copy(acc.at[pl.ds(s * RS, RS)], out_hbm.at[c, pl.ds(s * RS, RS)])


def _dis_of(degT):
    deg = 1.0 + degT[:, 0:1] + degT[:, 1:2]
    return jnp.where(deg > 0, lax.rsqrt(deg), 0.0)


def _tc1_body(degT_ref, x_ref, W1_ref, y_ref):
    y_ref[...] = _dis_of(degT_ref[...]) * jnp.dot(
        x_ref[...], W1_ref[...], preferred_element_type=jnp.float32,
        precision=lax.Precision.DEFAULT)


_tc1 = pl.pallas_call(
    _tc1_body,
    grid=(GRID_M,),
    in_specs=[
        pl.BlockSpec((MBLK, 2), lambda i: (i, 0)),
        pl.BlockSpec((MBLK, D_IN), lambda i: (i, 0)),
        pl.BlockSpec((D_IN, D), lambda i: (0, 0)),
    ],
    out_specs=pl.BlockSpec((MBLK, D), lambda i: (i, 0)),
    out_shape=jax.ShapeDtypeStruct((NPAD, D), jnp.float32),
)


def _tclayer_body(sp_ref, yp_ref, degT_ref, b_ref, W_ref, y_ref):
    dis = _dis_of(degT_ref[...])
    s = sp_ref[0] + sp_ref[1] + yp_ref[...]
    h = jax.nn.relu(dis * s + b_ref[...])
    y_ref[...] = dis * jnp.dot(h, W_ref[...], preferred_element_type=jnp.float32,
                               precision=lax.Precision.DEFAULT)


_tclayer = pl.pallas_call(
    _tclayer_body,
    grid=(GRID_M,),
    in_specs=[
        pl.BlockSpec((NC, MBLK, D), lambda i: (0, i, 0)),
        pl.BlockSpec((MBLK, D), lambda i: (i, 0)),
        pl.BlockSpec((MBLK, 2), lambda i: (i, 0)),
        pl.BlockSpec((1, D), lambda i: (0, 0)),
        pl.BlockSpec((D, D), lambda i: (0, 0)),
    ],
    out_specs=pl.BlockSpec((MBLK, D), lambda i: (i, 0)),
    out_shape=jax.ShapeDtypeStruct((NPAD, D), jnp.float32),
)


def _tcfinal_body(sp_ref, yp_ref, degT_ref, b5_ref, fc1W_ref, fc1b_ref,
                  fc2W_ref, fc2b_ref, o_ref):
    dis = _dis_of(degT_ref[...])
    s = sp_ref[0] + sp_ref[1] + yp_ref[...]
    h = jax.nn.relu(dis * s + b5_ref[...])
    h = jax.nn.relu(jnp.dot(h, fc1W_ref[...], preferred_element_type=jnp.float32,
                            precision=lax.Precision.DEFAULT) + fc1b_ref[...])
    o_ref[...] = jnp.dot(h, fc2W_ref[...], preferred_element_type=jnp.float32,
                         precision=lax.Precision.DEFAULT) + fc2b_ref[...]


_tcfinal = pl.pallas_call(
    _tcfinal_body,
    grid=(GRID_M,),
    in_specs=[
        pl.BlockSpec((NC, MBLK, D), lambda i: (0, i, 0)),
        pl.BlockSpec((MBLK, D), lambda i: (i, 0)),
        pl.BlockSpec((MBLK, 2), lambda i: (i, 0)),
        pl.BlockSpec((1, D), lambda i: (0, 0)),
        pl.BlockSpec((D, D), lambda i: (0, 0)),
        pl.BlockSpec((1, D), lambda i: (0, 0)),
        pl.BlockSpec((D, 8), lambda i: (0, 0)),
        pl.BlockSpec((1, 8), lambda i: (0, 0)),
    ],
    out_specs=pl.BlockSpec((MBLK, 8), lambda i: (i, 0)),
    out_shape=jax.ShapeDtypeStruct((NPAD, 8), jnp.float32),
)


def kernel(x, edge_index, edge_attr, W1, b1, W2, b2, W3, b3, W4, b4, W5, b5,
           fc1_W, fc1_b, fc2_W, fc2_b):
    src = edge_index[0].astype(jnp.int32)
    dst = edge_index[1].astype(jnp.int32)
    ew = edge_attr.astype(jnp.float32)

    npad_rows = E_PAD - E
    padidx = N + (jnp.arange(npad_rows, dtype=jnp.int32) % (NPAD - N))
    srcp = jnp.concatenate([src, padidx])
    dstp = jnp.concatenate([dst, padidx])
    ewp = jnp.concatenate([ew, jnp.zeros((npad_rows,), jnp.float32)])
    xp = jnp.pad(x, ((0, NPAD - N), (0, 0)))

    src3 = srcp.reshape(NW, NB * B)
    dst3 = dstp.reshape(NW, NB, B)
    ew3 = ewp.reshape(NW, NB * B)

    deg01 = _deg(dst3, ew3)
    degT = deg01.T
    y = _tc1(degT, xp, W1)

    bs = [b1.reshape(1, D), b2.reshape(1, D), b3.reshape(1, D),
          b4.reshape(1, D), b5.reshape(1, D)]
    Ws = [W2, W3, W4, W5]

    for li in range(4):
        sp = _agg(y, src3, dst3, ew3)
        y = _tclayer(sp, y, degT, bs[li], Ws[li])
    sp = _agg(y, src3, dst3, ew3)

    fc2_Wp = jnp.pad(fc2_W, ((0, 0), (0, 8 - fc2_W.shape[1])))
    fc2_bp = jnp.pad(fc2_b.reshape(1, -1), ((0, 0), (0, 8 - fc2_b.shape[0])))
    out = _tcfinal(sp, y, degT, bs[4], fc1_W,
                   fc1_b.reshape(1, D), fc2_Wp, fc2_bp)
    return out[:N, :1]
